# baseline (speedup 1.0000x reference)
import jax
import jax.numpy as jnp
from jax.experimental import pallas as pl

N = 10000
E = 320000
IN_DIM = 2
HID = 128
HEADS = 4
HD = HID // HEADS
SEQ = 12
INTERVAL = 4
SCALE = 1.0


def _layer_norm(x):
    mu = jnp.mean(x, axis=-1, keepdims=True)
    var = jnp.var(x, axis=-1, keepdims=True)
    return (x - mu) / jnp.sqrt(var + 1e-5)


def _gat(x, edge_index, Wg, a_src, a_dst):
    h = (x @ Wg).reshape(N, HEADS, HD)
    src = edge_index[0]
    dst = edge_index[1]
    e = jnp.sum(h[src] * a_src[None, :, :], axis=-1) + jnp.sum(h[dst] * a_dst[None, :, :], axis=-1)
    e = jax.nn.leaky_relu(e, 0.2)
    emax = jax.ops.segment_max(e, dst, num_segments=N)
    emax = jnp.where(jnp.isfinite(emax), emax, 0.0)
    ee = jnp.exp(e - emax[dst])
    esum = jax.ops.segment_sum(ee, dst, num_segments=N)
    alpha = ee / (esum[dst] + 1e-16)
    msg = h[src] * alpha[:, :, None]
    out = jax.ops.segment_sum(msg, dst, num_segments=N).reshape(N, HID)
    return jax.nn.elu(out)


def _ode(x, edge_index, Wg, a_src, a_dst):
    T = jnp.linspace(0.0, 1.0, INTERVAL + 1) * SCALE
    for i in range(INTERVAL):
        dt = T[i + 1] - T[i]
        x = x + dt * _gat(x, edge_index, Wg, a_src, a_dst)
    return x


def _final_proj_kernel(ret_ref, o1_ref, bo1_ref, o2_ref, bo2_ref, out_ref):
    t = jnp.tanh(ret_ref[...] @ o1_ref[...] + bo1_ref[...])
    out_ref[...] = t @ o2_ref[...] + bo2_ref[...]


def _final_proj(ret, O1, bo1, O2, bo2):
    BLK = 1000
    return pl.pallas_call(
        _final_proj_kernel,
        grid=(N // BLK,),
        in_specs=[
            pl.BlockSpec((BLK, HID), lambda i: (i, 0)),
            pl.BlockSpec((HID, HID), lambda i: (0, 0)),
            pl.BlockSpec((1, HID), lambda i: (0, 0)),
            pl.BlockSpec((HID, HID), lambda i: (0, 0)),
            pl.BlockSpec((1, HID), lambda i: (0, 0)),
        ],
        out_specs=pl.BlockSpec((BLK, HID), lambda i: (i, 0)),
        out_shape=jax.ShapeDtypeStruct((N, HID), jnp.float32),
    )(ret, O1, bo1.reshape(1, HID), O2, bo2.reshape(1, HID))


def kernel(inputs, edge_index, W_in, b_in, Wg, a_src, a_dst, Ww, bw, Uw, bu, O1, bo1, O2, bo2):
    x = inputs @ W_in + b_in
    ret = x[0, 0]
    for idx in range(SEQ):
        ret = _ode(ret, edge_index, Wg, a_src, a_dst)
        if idx != 0:
            inp = x[0, idx]
            ret = jnp.tanh(ret @ Ww + bw + inp @ Uw + bu)
        ret = _layer_norm(ret)
    out = _final_proj(ret, O1, bo1, O2, bo2)
    out = out.reshape(1, N, HID)[None]
    out = jnp.transpose(out, (1, 0, 2, 3))
    return out


# speedup vs baseline: 94.9640x; 94.9640x over previous
import jax
import jax.numpy as jnp
from jax import lax
from jax.experimental import pallas as pl
from jax.experimental.pallas import tpu as pltpu
from jax.experimental.pallas import tpu_sc as plsc

N = 10000
E = 320000
IN_DIM = 2
HID = 128
HEADS = 4
HD = HID // HEADS
SEQ = 12
INTERVAL = 4
DT = 0.25

NP = 10240
NC = 2
NS = 16
L = 16

HHALF = HID // NC
ACCW = 80
EPT = E // NS
CHUNK = 80
NCHUNK = EPT // CHUNK
GROUPS = CHUNK // L
RPT = NP // NS

_F32 = jnp.float32
_I32 = jnp.int32

_GDN = lax.GatherDimensionNumbers(
    offset_dims=(), collapsed_slice_dims=(0,), start_index_map=(0,))


def _vsplat(vec, i):
    idx = jnp.full((L, 1), i, _I32)
    return lax.gather(vec, idx, _GDN, (1,),
                      mode=lax.GatherScatterMode.PROMISE_IN_BOUNDS)


def _gat_sc_body(h2p, s8f, src_e, dst_e, accp, acc, s4t, gba, gbb, rb,
                 sba, sb2a, dba, sbb, sb2b, dbb, sema, semb):
    c = lax.axis_index("c")
    t = lax.axis_index("s")
    row0 = t * RPT
    lane = lax.iota(_I32, L)
    coff = c * NP

    def _zrow(r, carry):
        for k in range(ACCW // L):
            rb[r, pl.ds(k * L, L)] = jnp.zeros((L,), _F32)
        return carry
    lax.fori_loop(0, CHUNK, _zrow, 0)
    for zc in range(RPT // CHUNK):
        pltpu.sync_copy(rb, acc.at[pl.ds(row0 + zc * CHUNK, CHUNK)])
    pltpu.sync_copy(s8f.at[pl.ds((2 * c) * NP, 2 * NP)], s4t.at[pl.ds(0, 2 * NP)])
    pltpu.sync_copy(s8f.at[pl.ds((HEADS + 2 * c) * NP, 2 * NP)],
                    s4t.at[pl.ds(2 * NP, 2 * NP)])
    plsc.subcore_barrier()

    def _load(ci, sbx, sb2x, dbx):
        ebase = t * EPT + ci * CHUNK
        pltpu.sync_copy(src_e.at[pl.ds(ebase, CHUNK)], sbx)
        pltpu.sync_copy(dst_e.at[pl.ds(ebase, CHUNK)], dbx)
        for g in range(GROUPS):
            sb2x[pl.ds(g * L, L)] = sbx[pl.ds(g * L, L)] + coff

    def _proc(gbx, sbx, dbx):
        for g in range(GROUPS):
            sv = sbx[pl.ds(g * L, L)]
            dv = dbx[pl.ds(g * L, L)]
            ees = []
            for j in range(2):
                ss = plsc.load_gather(s4t, [sv + (j * NP)])
                sd = plsc.load_gather(s4t, [dv + ((2 + j) * NP)])
                e = ss + sd
                e = jnp.maximum(e, e * 0.2)
                ees.append(jnp.exp(e))
            for i in range(L):
                ei = g * L + i
                sp0 = _vsplat(ees[0], i)
                sp1 = _vsplat(ees[1], i)
                rb[ei, pl.ds(0, L)] = gbx[ei, pl.ds(0, L)] * sp0
                rb[ei, pl.ds(L, L)] = gbx[ei, pl.ds(L, L)] * sp0
                rb[ei, pl.ds(2 * L, L)] = gbx[ei, pl.ds(2 * L, L)] * sp1
                rb[ei, pl.ds(3 * L, L)] = gbx[ei, pl.ds(3 * L, L)] * sp1
                rb[ei, pl.ds(HHALF, L)] = jnp.where(lane == 0, sp0, sp1)
        pltpu.sync_copy(rb, acc.at[dbx], add=True)

    _load(0, sba, sb2a, dba)
    pltpu.async_copy(h2p.at[sb2a], gba, sema)

    def _epair(k, carry):
        _load(2 * k + 1, sbb, sb2b, dbb)
        pltpu.async_copy(h2p.at[sb2b], gbb, semb)
        pltpu.make_async_copy(h2p.at[sb2a], gba, sema).wait()
        _proc(gba, sba, dba)
        _load(lax.rem(2 * k + 2, NCHUNK), sba, sb2a, dba)
        pltpu.async_copy(h2p.at[sb2a], gba, sema)
        pltpu.make_async_copy(h2p.at[sb2b], gbb, semb).wait()
        _proc(gbb, sbb, dbb)
        return carry

    lax.fori_loop(0, NCHUNK // 2, _epair, 0)
    pltpu.make_async_copy(h2p.at[sb2a], gba, sema).wait()
    plsc.subcore_barrier()

    def _pchunk(pc, carry):
        r0 = row0 + pc * CHUNK
        pltpu.sync_copy(acc.at[pl.ds(r0, CHUNK)], rb)
        pltpu.sync_copy(rb, accp.at[c, pl.ds(r0, CHUNK)])
        return carry

    lax.fori_loop(0, RPT // CHUNK, _pchunk, 0)


_gat_sc = pl.kernel(
    _gat_sc_body,
    out_type=jax.ShapeDtypeStruct((NC, NP, ACCW), _F32),
    mesh=plsc.VectorSubcoreMesh(core_axis_name="c", subcore_axis_name="s"),
    compiler_params=pltpu.CompilerParams(needs_layout_passes=False),
    scratch_types=[
        pltpu.VMEM_SHARED((NP, ACCW), _F32),
        pltpu.VMEM((4 * NP,), _F32),
        pltpu.VMEM((CHUNK, HID), _F32),
        pltpu.VMEM((CHUNK, HID), _F32),
        pltpu.VMEM((CHUNK, ACCW), _F32),
        pltpu.VMEM((CHUNK,), _I32),
        pltpu.VMEM((CHUNK,), _I32),
        pltpu.VMEM((CHUNK,), _I32),
        pltpu.VMEM((CHUNK,), _I32),
        pltpu.VMEM((CHUNK,), _I32),
        pltpu.VMEM((CHUNK,), _I32),
        pltpu.SemaphoreType.DMA,
        pltpu.SemaphoreType.DMA,
    ],
)


BLK = 2048
GRID = NP // BLK

_acc_spec = pl.BlockSpec((NC, BLK, ACCW), lambda i: (0, i, 0))
_h2p_spec = pl.BlockSpec((NC, BLK, HID), lambda i: (0, i, 0))
_full_spec = pl.BlockSpec((BLK, HID), lambda i: (i, 0))
_w_spec = pl.BlockSpec((HID, HID), lambda i: (0, 0))
_a_spec = pl.BlockSpec((HID, 2 * HEADS), lambda i: (0, 0))
_p_spec = pl.BlockSpec((HEADS, HID), lambda i: (0, 0))
_b_spec = pl.BlockSpec((1, HID), lambda i: (0, 0))
_s8_spec = pl.BlockSpec((2 * HEADS, BLK), lambda i: (0, i))

_s8_shape = jax.ShapeDtypeStruct((2 * HEADS, NP), _F32)
_x_shape = jax.ShapeDtypeStruct((NP, HID), _F32)
_h2p_shape = jax.ShapeDtypeStruct((NC, NP, HID), _F32)


def _ln(x):
    mu = jnp.mean(x, axis=-1, keepdims=True)
    var = jnp.var(x, axis=-1, keepdims=True)
    return (x - mu) / jnp.sqrt(var + 1e-5)


def _finish(acc_ref, x_ref, pmat_ref):
    a0 = acc_ref[0]
    a1 = acc_ref[1]
    num = jnp.concatenate([a0[:, :HHALF], a1[:, :HHALF]], axis=1)
    es = jnp.concatenate([a0[:, HHALF:HHALF + 2], a1[:, HHALF:HHALF + 2]],
                         axis=1)
    esb = jnp.dot(es, pmat_ref[...], preferred_element_type=_F32)
    v = num / (esb + 1e-16)
    o = jnp.where(v > 0.0, v, jnp.exp(jnp.minimum(v, 0.0)) - 1.0)
    return x_ref[...] + DT * o


def _emit_prep(x, wg_ref, ab_ref, h_ref, s8_ref):
    h = jnp.dot(x, wg_ref[...], preferred_element_type=_F32)
    h_ref[0] = h
    h_ref[1] = jnp.concatenate([h[:, HHALF:], h[:, :HHALF]], axis=1)
    s8_ref[...] = lax.dot_general(ab_ref[...], h, (((0,), (1,)), ((), ())),
                                  preferred_element_type=_F32)


def _init_prep_body(x0_ref, wg_ref, ab_ref, h_ref, s8_ref):
    _emit_prep(x0_ref[...], wg_ref, ab_ref, h_ref, s8_ref)


_init_prep = pl.pallas_call(
    _init_prep_body,
    grid=(GRID,),
    in_specs=[_full_spec, _w_spec, _a_spec],
    out_specs=[_h2p_spec, _s8_spec],
    out_shape=[_h2p_shape, _s8_shape],
)


def _step_fin_body(acc_ref, x_ref, wg_ref, ab_ref, pmat_ref,
                   xn_ref, h_ref, s8_ref):
    xn = _finish(acc_ref, x_ref, pmat_ref)
    xn_ref[...] = xn
    _emit_prep(xn, wg_ref, ab_ref, h_ref, s8_ref)


_step_fin = pl.pallas_call(
    _step_fin_body,
    grid=(GRID,),
    in_specs=[_acc_spec, _full_spec, _w_spec, _a_spec, _p_spec],
    out_specs=[_full_spec, _h2p_spec, _s8_spec],
    out_shape=[_x_shape, _h2p_shape, _s8_shape],
)


def _ln_fin_body(acc_ref, x_ref, wg_ref, ab_ref, pmat_ref,
                 xn_ref, h_ref, s8_ref):
    xn = _ln(_finish(acc_ref, x_ref, pmat_ref))
    xn_ref[...] = xn
    _emit_prep(xn, wg_ref, ab_ref, h_ref, s8_ref)


_ln_fin = pl.pallas_call(
    _ln_fin_body,
    grid=(GRID,),
    in_specs=[_acc_spec, _full_spec, _w_spec, _a_spec, _p_spec],
    out_specs=[_full_spec, _h2p_spec, _s8_spec],
    out_shape=[_x_shape, _h2p_shape, _s8_shape],
)


def _mix_fin_body(acc_ref, x_ref, inp_ref, ww_ref, uw_ref, bwu_ref,
                  wg_ref, ab_ref, pmat_ref, xn_ref, h_ref, s8_ref):
    xn = _finish(acc_ref, x_ref, pmat_ref)
    xn = jnp.tanh(jnp.dot(xn, ww_ref[...], preferred_element_type=_F32)
                  + jnp.dot(inp_ref[...], uw_ref[...], preferred_element_type=_F32)
                  + bwu_ref[...])
    xn = _ln(xn)
    xn_ref[...] = xn
    _emit_prep(xn, wg_ref, ab_ref, h_ref, s8_ref)


_mix_fin = pl.pallas_call(
    _mix_fin_body,
    grid=(GRID,),
    in_specs=[_acc_spec, _full_spec, _full_spec, _w_spec, _w_spec, _b_spec,
              _w_spec, _a_spec, _p_spec],
    out_specs=[_full_spec, _h2p_spec, _s8_spec],
    out_shape=[_x_shape, _h2p_shape, _s8_shape],
)


def _final_fin_body(acc_ref, x_ref, inp_ref, ww_ref, uw_ref, bwu_ref,
                    o1_ref, bo1_ref, o2_ref, bo2_ref, pmat_ref, out_ref):
    xn = _finish(acc_ref, x_ref, pmat_ref)
    xn = jnp.tanh(jnp.dot(xn, ww_ref[...], preferred_element_type=_F32)
                  + jnp.dot(inp_ref[...], uw_ref[...], preferred_element_type=_F32)
                  + bwu_ref[...])
    xn = _ln(xn)
    tt = jnp.tanh(jnp.dot(xn, o1_ref[...], preferred_element_type=_F32)
                  + bo1_ref[...])
    out_ref[...] = jnp.dot(tt, o2_ref[...], preferred_element_type=_F32) + bo2_ref[...]


_final_fin = pl.pallas_call(
    _final_fin_body,
    grid=(GRID,),
    in_specs=[_acc_spec, _full_spec, _full_spec, _w_spec, _w_spec, _b_spec,
              _w_spec, _b_spec, _w_spec, _b_spec, _p_spec],
    out_specs=_full_spec,
    out_shape=_x_shape,
)

XIN_PAD = 8


def _xseq_body(in_ref, w_ref, b_ref, out_ref):
    out_ref[...] = (jnp.dot(in_ref[...], w_ref[...], preferred_element_type=_F32)
                    + b_ref[...])


_xseq = pl.pallas_call(
    _xseq_body,
    grid=(SEQ * NP // BLK,),
    in_specs=[pl.BlockSpec((BLK, XIN_PAD), lambda i: (i, 0)),
              pl.BlockSpec((XIN_PAD, HID), lambda i: (0, 0)),
              pl.BlockSpec((1, HID), lambda i: (0, 0))],
    out_specs=pl.BlockSpec((BLK, HID), lambda i: (i, 0)),
    out_shape=jax.ShapeDtypeStruct((SEQ * NP, HID), _F32),
)


def kernel(inputs, edge_index, W_in, b_in, Wg, a_src, a_dst, Ww, bw, Uw, bu,
           O1, bo1, O2, bo2):
    src_e = edge_index[0]
    dst_e = edge_index[1]

    asf = a_src.reshape(HID)
    adf = a_dst.reshape(HID)
    head = jnp.arange(HID, dtype=_I32) // HD
    eye = (head[:, None] == jnp.arange(HEADS, dtype=_I32)[None, :]).astype(_F32)
    ablk = jnp.concatenate([asf[:, None] * eye, adf[:, None] * eye], axis=1)
    pmat = eye.T

    inp3 = inputs.reshape(SEQ, N, IN_DIM)
    inp_pad = jnp.pad(inp3, ((0, 0), (0, NP - N), (0, XIN_PAD - IN_DIM)))
    inp_pad = inp_pad.reshape(SEQ * NP, XIN_PAD)
    w_in_pad = jnp.pad(W_in, ((0, XIN_PAD - IN_DIM), (0, 0)))
    xseq = _xseq(inp_pad, w_in_pad, b_in.reshape(1, HID)).reshape(SEQ, NP, HID)

    bwu = (bw + bu).reshape(1, HID)
    bo1r = bo1.reshape(1, HID)
    bo2r = bo2.reshape(1, HID)

    x = xseq[0]
    h2p, s8t = _init_prep(x, Wg, ablk)
    out = None
    for idx in range(SEQ):
        for step in range(INTERVAL):
            accp = _gat_sc(h2p.reshape(NC * NP, HID),
                           s8t.reshape(2 * HEADS * NP), src_e, dst_e)
            if step < INTERVAL - 1:
                x, h2p, s8t = _step_fin(accp, x, Wg, ablk, pmat)
            elif idx == 0:
                x, h2p, s8t = _ln_fin(accp, x, Wg, ablk, pmat)
            elif idx < SEQ - 1:
                x, h2p, s8t = _mix_fin(accp, x, xseq[idx], Ww, Uw, bwu, Wg,
                                       ablk, pmat)
            else:
                out = _final_fin(accp, x, xseq[idx], Ww, Uw, bwu,
                                 O1, bo1r, O2, bo2r, pmat)
    return out[:N].reshape(1, 1, N, HID)
